# baseline (speedup 1.0000x reference)
import jax
import jax.numpy as jnp
from jax import lax
from jax.experimental import pallas as pl
from jax.experimental.pallas import tpu as pltpu
from jax.experimental.pallas import tpu_sc as plsc

N_USERS_C = 500000
B_C = 16384
D_C = 16
ROWS_PER_BLK = 8
BLK_W = ROWS_PER_BLK * D_C
NUM_CORES = 2
NUM_SUBCORES = 16
NW = NUM_CORES * NUM_SUBCORES
CHUNK = B_C // NW
HALF = CHUNK // 2


def _sc_body(tbl_hbm, users_hbm, items_hbm, w_hbm,
             scores_hbm, l2_hbm,
             u_idx, i_idx, ub_idx, ib_idx, u_blk, i_blk, w_v,
             s_out, l_out, sem):
    wid = lax.axis_index("s") * NUM_CORES + lax.axis_index("c")
    base = wid * CHUNK

    pltpu.sync_copy(users_hbm.at[pl.ds(base, CHUNK)], u_idx)
    pltpu.sync_copy(items_hbm.at[pl.ds(base, CHUNK)], i_idx)
    pltpu.sync_copy(w_hbm, w_v)

    iota = lax.iota(jnp.int32, D_C)
    w_vec = w_v[...]
    _dn = lax.GatherDimensionNumbers(
        offset_dims=(), collapsed_slice_dims=(0,), start_index_map=(0,))
    w_bcast = [
        lax.gather(w_vec, jnp.full((D_C, 1), d, jnp.int32),
                   dimension_numbers=_dn, slice_sizes=(1,),
                   mode=lax.GatherScatterMode.PROMISE_IN_BOUNDS)
        for d in range(D_C)
    ]

    for h in range(2):
        off = h * HALF
        for k in range(HALF // D_C):
            sl_src = pl.ds(off + k * D_C, D_C)
            sl_dst = pl.ds(k * D_C, D_C)
            ub_idx[sl_dst] = lax.shift_right_logical(u_idx[sl_src], 3)
            ib_idx[sl_dst] = lax.shift_right_logical(i_idx[sl_src], 3) + (
                N_USERS_C // ROWS_PER_BLK)

        cp_u = pltpu.make_async_copy(tbl_hbm.at[ub_idx], u_blk, sem)
        cp_i = pltpu.make_async_copy(tbl_hbm.at[ib_idx], i_blk, sem)
        cp_u.start()
        cp_i.start()
        cp_u.wait()
        cp_i.wait()

        def group(g, carry):
            rows = g * D_C + iota
            u16 = u_idx[pl.ds(off + g * D_C, D_C)]
            i16 = i_idx[pl.ds(off + g * D_C, D_C)]
            u_sub = lax.shift_left(jnp.bitwise_and(u16, 7), 4)
            i_sub = lax.shift_left(jnp.bitwise_and(i16, 7), 4)
            acc_s = jnp.zeros((D_C,), jnp.float32)
            acc_l = jnp.zeros((D_C,), jnp.float32)
            for d in range(D_C):
                u = plsc.load_gather(u_blk, [rows, u_sub + d])
                it = plsc.load_gather(i_blk, [rows, i_sub + d])
                s = u * it * w_bcast[d]
                acc_s = acc_s + s
                acc_l = acc_l + s * s
            s_out[pl.ds(off + g * D_C, D_C)] = acc_s
            l_out[pl.ds(off + g * D_C, D_C)] = acc_l
            return carry

        lax.fori_loop(0, HALF // D_C, group, 0)

    pltpu.sync_copy(s_out, scores_hbm.at[pl.ds(base, CHUNK)])
    pltpu.sync_copy(l_out, l2_hbm.at[pl.ds(base, CHUNK)])


def kernel(mf_table, mlp_table, W1, b1, out_w, users, items):
    users = users.astype(jnp.int32)
    items = items.astype(jnp.int32)
    w16 = out_w[0, :D_C].astype(jnp.float32)
    tbl = mf_table.reshape(-1, BLK_W)

    mesh = plsc.VectorSubcoreMesh(core_axis_name="c", subcore_axis_name="s")
    scores, l2 = pl.kernel(
        _sc_body,
        out_type=(
            jax.ShapeDtypeStruct((B_C,), jnp.float32),
            jax.ShapeDtypeStruct((B_C,), jnp.float32),
        ),
        mesh=mesh,
        compiler_params=pltpu.CompilerParams(needs_layout_passes=False),
        scratch_types=[
            pltpu.VMEM((CHUNK,), jnp.int32),
            pltpu.VMEM((CHUNK,), jnp.int32),
            pltpu.VMEM((HALF,), jnp.int32),
            pltpu.VMEM((HALF,), jnp.int32),
            pltpu.VMEM((HALF, BLK_W), jnp.float32),
            pltpu.VMEM((HALF, BLK_W), jnp.float32),
            pltpu.VMEM((D_C,), jnp.float32),
            pltpu.VMEM((CHUNK,), jnp.float32),
            pltpu.VMEM((CHUNK,), jnp.float32),
            pltpu.SemaphoreType.DMA,
        ],
    )(tbl, users, items, w16)
    return (scores, l2)
